# baseline (speedup 1.0000x reference)
import functools

import jax
import jax.numpy as jnp
from jax import lax
from jax.experimental import pallas as pl
from jax.experimental.pallas import tpu as pltpu
from jax.experimental.pallas import tpu_sc as plsc

_E = 8
_TOPK = 2
_H = 2048
_I = 1024
_T = 4096

_B = 384
_A = _T * _TOPK
_S = ((_A + _E * (_B - 1) + _B - 1) // _B) * _B
_NB = _S // _B

_NC = 2
_NS = 16
_NW = _NC * _NS

_FR = 64
_FC = 128


def _route_body(fe_ref, slot_ref, be_ref):
  fe = fe_ref[...]
  fef = fe.astype(jnp.float32)
  rr = lax.broadcasted_iota(jnp.int32, (_FC, _FC), 0)
  cc = lax.broadcasted_iota(jnp.int32, (_FC, _FC), 1)
  upper = (rr < cc).astype(jnp.float32)
  r2 = lax.broadcasted_iota(jnp.int32, (_FR, _FR), 0)
  c2 = lax.broadcasted_iota(jnp.int32, (_FR, _FR), 1)
  lower = (c2 < r2).astype(jnp.float32)
  rank = jnp.zeros((_FR, _FC), jnp.float32)
  gmap = jnp.zeros((_FR, _FC), jnp.float32)
  gs = jnp.zeros((), jnp.float32)
  bvec = lax.broadcasted_iota(
      jnp.int32, (1, _FC), 1).astype(jnp.float32) * float(_B)
  be = jnp.zeros((1, _FC), jnp.float32)
  for e in range(_E):
    m = (fef == float(e)).astype(jnp.float32)
    p = jnp.dot(m, upper, preferred_element_type=jnp.float32)
    t = jnp.sum(m, axis=1, keepdims=True)
    o = jnp.dot(lower, t, preferred_element_type=jnp.float32)
    rank = rank + m * (p + o)
    gmap = gmap + m * gs
    cnt = jnp.sum(m)
    padded = jnp.ceil(cnt / float(_B)) * float(_B)
    gs = gs + padded
    be = be + (bvec >= gs).astype(jnp.float32)
  slot_ref[...] = (rank + gmap).astype(jnp.int32)
  be_ref[...] = jnp.minimum(be, float(_E - 1)).astype(jnp.int32)


def _route(expert_indices):
  fe2d = expert_indices.astype(jnp.int32).reshape(_FR, _FC)
  slotmat, bemat = pl.pallas_call(
      _route_body,
      out_shape=(jax.ShapeDtypeStruct((_FR, _FC), jnp.int32),
                 jax.ShapeDtypeStruct((1, _FC), jnp.int32)),
  )(fe2d)
  slotv = slotmat.reshape(_NW, 2 * _T // _NW)
  block_expert = bemat[0, :_NB].copy()
  return block_expert, slotv


_D_CH = 16


def _sc_dispatch(x, slotv):
  tok_per_w = _T // _NW
  n_ch = tok_per_w // _D_CH
  mesh = plsc.VectorSubcoreMesh(core_axis_name="c", subcore_axis_name="s")

  @functools.partial(
      pl.kernel,
      out_type=jax.ShapeDtypeStruct((_S, _H), jnp.float32),
      mesh=mesh,
      scratch_types=[
          pltpu.VMEM((2 * tok_per_w,), jnp.int32),
          pltpu.VMEM((n_ch, _D_CH), jnp.int32),
          pltpu.VMEM((n_ch, _D_CH), jnp.int32),
          pltpu.VMEM((_D_CH, _H), jnp.float32),
          pltpu.VMEM((_D_CH, _H), jnp.float32),
          pltpu.SemaphoreType.DMA,
          pltpu.SemaphoreType.DMA,
          pltpu.SemaphoreType.DMA,
          pltpu.SemaphoreType.DMA,
          pltpu.SemaphoreType.DMA,
          pltpu.SemaphoreType.DMA,
      ],
      compiler_params=pltpu.CompilerParams(needs_layout_passes=False),
  )
  def dispatch_k(x_hbm, p_hbm, xg_hbm, sv, i0, i1, xb0, xb1,
                 si0, si1, sa0, sa1, sb0, sb1):
    wid = lax.axis_index("s") * _NC + lax.axis_index("c")
    base = wid * tok_per_w
    pltpu.sync_copy(p_hbm.at[wid], sv)
    lanes = lax.iota(jnp.int32, 16)
    for j in range(n_ch):
      ev = lanes * 2 + j * 2 * _D_CH
      i0[j] = plsc.load_gather(sv, [ev])
      i1[j] = plsc.load_gather(sv, [ev + 1])
    xbufs, isems = (xb0, xb1), (si0, si1)
    osems = ((sa0, sb0), (sa1, sb1))
    outs = [None] * n_ch

    gin = pltpu.async_copy(x_hbm.at[pl.ds(base, _D_CH)], xb0, si0)
    for i in range(n_ch):
      p = i % 2
      gin.wait()
      if i + 1 < n_ch:
        if i >= 1:
          outs[i - 1][0].wait()
          outs[i - 1][1].wait()
        gin = pltpu.async_copy(
            x_hbm.at[pl.ds(base + (i + 1) * _D_CH, _D_CH)],
            xbufs[1 - p], isems[1 - p])
      outs[i] = (
          pltpu.async_copy(xbufs[p], xg_hbm.at[i0.at[i]], osems[p][0]),
          pltpu.async_copy(xbufs[p], xg_hbm.at[i1.at[i]], osems[p][1]),
      )
    outs[n_ch - 2][0].wait()
    outs[n_ch - 2][1].wait()
    outs[n_ch - 1][0].wait()
    outs[n_ch - 1][1].wait()

  return dispatch_k(x, slotv)


def _tc_gemm_body(be_ref, xg_ref, w1g_ref, w1u_ref, w2_ref, yg_ref):
  del be_ref
  xb = xg_ref[...]
  w1g = w1g_ref[0]
  w1u = w1u_ref[0]
  dn = (((1,), (1,)), ((), ()))
  gate = lax.dot_general(xb, w1g, dn, preferred_element_type=jnp.float32)
  up = lax.dot_general(xb, w1u, dn, preferred_element_type=jnp.float32)
  act = gate * jax.nn.sigmoid(gate) * up
  w2c = w2_ref[0]
  yg_ref[...] = lax.dot_general(act, w2c, (((1,), (1,)), ((), ())),
                                preferred_element_type=jnp.float32)


def _tc_gemm(xg, w1, w2, block_expert):
  spec = pltpu.PrefetchScalarGridSpec(
      num_scalar_prefetch=1,
      grid=(_NB,),
      in_specs=[
          pl.BlockSpec((_B, _H), lambda b, be: (b, 0)),
          pl.BlockSpec((1, _I, _H), lambda b, be: (be[b], 0, 0)),
          pl.BlockSpec((1, _I, _H), lambda b, be: (be[b], 1, 0)),
          pl.BlockSpec((1, _H, _I), lambda b, be: (be[b], 0, 0)),
      ],
      out_specs=pl.BlockSpec((_B, _H), lambda b, be: (b, 0)),
  )
  return pl.pallas_call(
      _tc_gemm_body,
      grid_spec=spec,
      out_shape=jax.ShapeDtypeStruct((_S, _H), jnp.float32),
      compiler_params=pltpu.CompilerParams(
          dimension_semantics=("arbitrary",),
          vmem_limit_bytes=100 * 1024 * 1024,
      ),
  )(block_expert, xg, w1, w1, w2)


_C_CH = 8


def _sc_combine(yg, slotv, ewv):
  tok_per_w = _T // _NW
  n_ch = tok_per_w // _C_CH
  mesh = plsc.VectorSubcoreMesh(core_axis_name="c", subcore_axis_name="s")

  @functools.partial(
      pl.kernel,
      out_type=jax.ShapeDtypeStruct((_T, _H), jnp.float32),
      mesh=mesh,
      scratch_types=[
          pltpu.VMEM((2 * tok_per_w,), jnp.int32),
          pltpu.VMEM((8, 16), jnp.int32),
          pltpu.VMEM((8, 16), jnp.int32),
          pltpu.VMEM((2 * tok_per_w,), jnp.float32),
          pltpu.VMEM((_C_CH, _H), jnp.float32),
          pltpu.VMEM((_C_CH, _H), jnp.float32),
          pltpu.VMEM((_C_CH, _H), jnp.float32),
          pltpu.VMEM((_C_CH, _H), jnp.float32),
          pltpu.SemaphoreType.DMA,
          pltpu.SemaphoreType.DMA,
          pltpu.SemaphoreType.DMA,
          pltpu.SemaphoreType.DMA,
          pltpu.SemaphoreType.DMA,
          pltpu.SemaphoreType.DMA,
      ],
      compiler_params=pltpu.CompilerParams(needs_layout_passes=False),
  )
  def combine_k(yg_hbm, p_hbm, ew_hbm, out_hbm, sv, i0, i1,
                wv, a0, b0, a1, b1, sa0, sb0, sa1, sb1, so0, so1):
    wid = lax.axis_index("s") * _NC + lax.axis_index("c")
    base = wid * tok_per_w
    pltpu.sync_copy(p_hbm.at[wid], sv)
    pltpu.sync_copy(ew_hbm.at[wid], wv)
    lanes = lax.iota(jnp.int32, 16)
    for j in range(tok_per_w // 16):
      ev = lanes * 2 + j * 32
      i0[j] = plsc.load_gather(sv, [ev])
      i1[j] = plsc.load_gather(sv, [ev + 1])
    abufs, bbufs = (a0, a1), (b0, b1)
    asems, bsems, osems = (sa0, sa1), (sb0, sb1), (so0, so1)
    outs = [None] * n_ch

    def start(i):
      p = i % 2
      sl = (i // 2, pl.ds((i % 2) * _C_CH, _C_CH))
      ca = pltpu.async_copy(yg_hbm.at[i0.at[sl]], abufs[p], asems[p])
      cb = pltpu.async_copy(yg_hbm.at[i1.at[sl]], bbufs[p], bsems[p])
      return ca, cb

    g = start(0)
    for i in range(n_ch):
      p = i % 2
      g[0].wait()
      g[1].wait()
      if i + 1 < n_ch:
        if i >= 1:
          outs[i - 1].wait()
        g = start(i + 1)
      ra, rb = abufs[p], bbufs[p]

      def add_row(r, carry, ra=ra, rb=rb, i=i):
        tloc = jnp.full((16,), 2 * (i * _C_CH + r), jnp.int32)
        w0 = plsc.load_gather(wv, [tloc])
        w1 = plsc.load_gather(wv, [tloc + 1])

        @plsc.parallel_loop(0, _H, 16, unroll=8)
        def wadd(j):
          sl = (r, pl.ds(j, 16))
          ra[sl] = ra[sl] * w0 + rb[sl] * w1
        return carry

      lax.fori_loop(0, _C_CH, add_row, 0)
      outs[i] = pltpu.async_copy(
          ra, out_hbm.at[pl.ds(base + i * _C_CH, _C_CH)], osems[p])
    outs[n_ch - 2].wait()
    outs[n_ch - 1].wait()

  return combine_k(yg, slotv, ewv)


def kernel(x, expert_weights, expert_indices, top_k, w1_weight, w2_weight):
  del top_k
  block_expert, slotv = _route(expert_indices)
  ewv = expert_weights.astype(jnp.float32).reshape(_NW, 2 * _T // _NW)
  xg = _sc_dispatch(x, slotv)
  yg = _tc_gemm(xg, w1_weight, w2_weight, block_expert)
  return _sc_combine(yg, slotv, ewv)


# speedup vs baseline: 1.0256x; 1.0256x over previous
import functools

import jax
import jax.numpy as jnp
from jax import lax
from jax.experimental import pallas as pl
from jax.experimental.pallas import tpu as pltpu
from jax.experimental.pallas import tpu_sc as plsc

_E = 8
_TOPK = 2
_H = 2048
_I = 1024
_T = 4096

_B = 256
_A = _T * _TOPK
_S = ((_A + _E * (_B - 1) + _B - 1) // _B) * _B
_NB = _S // _B

_NC = 2
_NS = 16
_NW = _NC * _NS

_FR = 64
_FC = 128


def _route_body(fe_ref, slot_ref, be_ref):
  fe = fe_ref[...]
  fef = fe.astype(jnp.float32)
  rr = lax.broadcasted_iota(jnp.int32, (_FC, _FC), 0)
  cc = lax.broadcasted_iota(jnp.int32, (_FC, _FC), 1)
  upper = (rr < cc).astype(jnp.float32)
  r2 = lax.broadcasted_iota(jnp.int32, (_FR, _FR), 0)
  c2 = lax.broadcasted_iota(jnp.int32, (_FR, _FR), 1)
  lower = (c2 < r2).astype(jnp.float32)
  rank = jnp.zeros((_FR, _FC), jnp.float32)
  gmap = jnp.zeros((_FR, _FC), jnp.float32)
  gs = jnp.zeros((), jnp.float32)
  bvec = lax.broadcasted_iota(
      jnp.int32, (1, _FC), 1).astype(jnp.float32) * float(_B)
  be = jnp.zeros((1, _FC), jnp.float32)
  for e in range(_E):
    m = (fef == float(e)).astype(jnp.float32)
    p = jnp.dot(m, upper, preferred_element_type=jnp.float32)
    t = jnp.sum(m, axis=1, keepdims=True)
    o = jnp.dot(lower, t, preferred_element_type=jnp.float32)
    rank = rank + m * (p + o)
    gmap = gmap + m * gs
    cnt = jnp.sum(m)
    padded = jnp.ceil(cnt / float(_B)) * float(_B)
    gs = gs + padded
    be = be + (bvec >= gs).astype(jnp.float32)
  slot_ref[...] = (rank + gmap).astype(jnp.int32)
  be_ref[...] = jnp.minimum(be, float(_E - 1)).astype(jnp.int32)


def _route(expert_indices):
  fe2d = expert_indices.astype(jnp.int32).reshape(_FR, _FC)
  slotmat, bemat = pl.pallas_call(
      _route_body,
      out_shape=(jax.ShapeDtypeStruct((_FR, _FC), jnp.int32),
                 jax.ShapeDtypeStruct((1, _FC), jnp.int32)),
  )(fe2d)
  slotv = slotmat.reshape(_NW, 2 * _T // _NW)
  block_expert = bemat[0, :_NB].copy()
  return block_expert, slotv


_D_CH = 16


def _sc_dispatch(x, slotv):
  tok_per_w = _T // _NW
  n_ch = tok_per_w // _D_CH
  mesh = plsc.VectorSubcoreMesh(core_axis_name="c", subcore_axis_name="s")

  @functools.partial(
      pl.kernel,
      out_type=jax.ShapeDtypeStruct((_S, _H), jnp.float32),
      mesh=mesh,
      scratch_types=[
          pltpu.VMEM((2 * tok_per_w,), jnp.int32),
          pltpu.VMEM((n_ch, _D_CH), jnp.int32),
          pltpu.VMEM((n_ch, _D_CH), jnp.int32),
          pltpu.VMEM((_D_CH, _H), jnp.float32),
          pltpu.VMEM((_D_CH, _H), jnp.float32),
          pltpu.SemaphoreType.DMA,
          pltpu.SemaphoreType.DMA,
          pltpu.SemaphoreType.DMA,
          pltpu.SemaphoreType.DMA,
          pltpu.SemaphoreType.DMA,
          pltpu.SemaphoreType.DMA,
      ],
      compiler_params=pltpu.CompilerParams(needs_layout_passes=False),
  )
  def dispatch_k(x_hbm, p_hbm, xg_hbm, sv, i0, i1, xb0, xb1,
                 si0, si1, sa0, sa1, sb0, sb1):
    wid = lax.axis_index("s") * _NC + lax.axis_index("c")
    base = wid * tok_per_w
    pltpu.sync_copy(p_hbm.at[wid], sv)
    lanes = lax.iota(jnp.int32, 16)
    for j in range(n_ch):
      ev = lanes * 2 + j * 2 * _D_CH
      i0[j] = plsc.load_gather(sv, [ev])
      i1[j] = plsc.load_gather(sv, [ev + 1])
    xbufs, isems = (xb0, xb1), (si0, si1)
    osems = ((sa0, sb0), (sa1, sb1))
    outs = [None] * n_ch

    gin = pltpu.async_copy(x_hbm.at[pl.ds(base, _D_CH)], xb0, si0)
    for i in range(n_ch):
      p = i % 2
      gin.wait()
      if i + 1 < n_ch:
        if i >= 1:
          outs[i - 1][0].wait()
          outs[i - 1][1].wait()
        gin = pltpu.async_copy(
            x_hbm.at[pl.ds(base + (i + 1) * _D_CH, _D_CH)],
            xbufs[1 - p], isems[1 - p])
      outs[i] = (
          pltpu.async_copy(xbufs[p], xg_hbm.at[i0.at[i]], osems[p][0]),
          pltpu.async_copy(xbufs[p], xg_hbm.at[i1.at[i]], osems[p][1]),
      )
    outs[n_ch - 2][0].wait()
    outs[n_ch - 2][1].wait()
    outs[n_ch - 1][0].wait()
    outs[n_ch - 1][1].wait()

  return dispatch_k(x, slotv)


def _tc_gemm_body(be_ref, xg_ref, w1g_ref, w1u_ref, w2_ref, yg_ref):
  del be_ref
  xb = xg_ref[...]
  w1g = w1g_ref[0]
  w1u = w1u_ref[0]
  dn = (((1,), (1,)), ((), ()))
  gate = lax.dot_general(xb, w1g, dn, preferred_element_type=jnp.float32)
  up = lax.dot_general(xb, w1u, dn, preferred_element_type=jnp.float32)
  act = gate * jax.nn.sigmoid(gate) * up
  w2c = w2_ref[0]
  yg_ref[...] = lax.dot_general(act, w2c, (((1,), (1,)), ((), ())),
                                preferred_element_type=jnp.float32)


def _tc_gemm(xg, w1, w2, block_expert):
  spec = pltpu.PrefetchScalarGridSpec(
      num_scalar_prefetch=1,
      grid=(_NB,),
      in_specs=[
          pl.BlockSpec((_B, _H), lambda b, be: (b, 0)),
          pl.BlockSpec((1, _I, _H), lambda b, be: (be[b], 0, 0)),
          pl.BlockSpec((1, _I, _H), lambda b, be: (be[b], 1, 0)),
          pl.BlockSpec((1, _H, _I), lambda b, be: (be[b], 0, 0)),
      ],
      out_specs=pl.BlockSpec((_B, _H), lambda b, be: (b, 0)),
  )
  return pl.pallas_call(
      _tc_gemm_body,
      grid_spec=spec,
      out_shape=jax.ShapeDtypeStruct((_S, _H), jnp.float32),
      compiler_params=pltpu.CompilerParams(
          dimension_semantics=("arbitrary",),
          vmem_limit_bytes=100 * 1024 * 1024,
      ),
  )(block_expert, xg, w1, w1, w2)


_C_CH = 8


def _sc_combine(yg, slotv, ewv):
  tok_per_w = _T // _NW
  n_ch = tok_per_w // _C_CH
  mesh = plsc.VectorSubcoreMesh(core_axis_name="c", subcore_axis_name="s")

  @functools.partial(
      pl.kernel,
      out_type=jax.ShapeDtypeStruct((_T, _H), jnp.float32),
      mesh=mesh,
      scratch_types=[
          pltpu.VMEM((2 * tok_per_w,), jnp.int32),
          pltpu.VMEM((8, 16), jnp.int32),
          pltpu.VMEM((8, 16), jnp.int32),
          pltpu.VMEM((2 * tok_per_w,), jnp.float32),
          pltpu.VMEM((_C_CH, _H), jnp.float32),
          pltpu.VMEM((_C_CH, _H), jnp.float32),
          pltpu.VMEM((_C_CH, _H), jnp.float32),
          pltpu.VMEM((_C_CH, _H), jnp.float32),
          pltpu.SemaphoreType.DMA,
          pltpu.SemaphoreType.DMA,
          pltpu.SemaphoreType.DMA,
          pltpu.SemaphoreType.DMA,
          pltpu.SemaphoreType.DMA,
          pltpu.SemaphoreType.DMA,
      ],
      compiler_params=pltpu.CompilerParams(needs_layout_passes=False),
  )
  def combine_k(yg_hbm, p_hbm, ew_hbm, out_hbm, sv, i0, i1,
                wv, a0, b0, a1, b1, sa0, sb0, sa1, sb1, so0, so1):
    wid = lax.axis_index("s") * _NC + lax.axis_index("c")
    base = wid * tok_per_w
    pltpu.sync_copy(p_hbm.at[wid], sv)
    pltpu.sync_copy(ew_hbm.at[wid], wv)
    lanes = lax.iota(jnp.int32, 16)
    for j in range(tok_per_w // 16):
      ev = lanes * 2 + j * 32
      i0[j] = plsc.load_gather(sv, [ev])
      i1[j] = plsc.load_gather(sv, [ev + 1])
    abufs, bbufs = (a0, a1), (b0, b1)
    asems, bsems, osems = (sa0, sa1), (sb0, sb1), (so0, so1)
    outs = [None] * n_ch

    def start(i):
      p = i % 2
      sl = (i // 2, pl.ds((i % 2) * _C_CH, _C_CH))
      ca = pltpu.async_copy(yg_hbm.at[i0.at[sl]], abufs[p], asems[p])
      cb = pltpu.async_copy(yg_hbm.at[i1.at[sl]], bbufs[p], bsems[p])
      return ca, cb

    g = start(0)
    for i in range(n_ch):
      p = i % 2
      g[0].wait()
      g[1].wait()
      if i + 1 < n_ch:
        if i >= 1:
          outs[i - 1].wait()
        g = start(i + 1)
      ra, rb = abufs[p], bbufs[p]

      def add_row(r, carry, ra=ra, rb=rb, i=i):
        tloc = jnp.full((16,), 2 * (i * _C_CH + r), jnp.int32)
        w0 = plsc.load_gather(wv, [tloc])
        w1 = plsc.load_gather(wv, [tloc + 1])

        @plsc.parallel_loop(0, _H, 16, unroll=8)
        def wadd(j):
          sl = (r, pl.ds(j, 16))
          ra[sl] = ra[sl] * w0 + rb[sl] * w1
        return carry

      lax.fori_loop(0, _C_CH, add_row, 0)
      outs[i] = pltpu.async_copy(
          ra, out_hbm.at[pl.ds(base + i * _C_CH, _C_CH)], osems[p])
    outs[n_ch - 2].wait()
    outs[n_ch - 1].wait()

  return combine_k(yg, slotv, ewv)


def kernel(x, expert_weights, expert_indices, top_k, w1_weight, w2_weight):
  del top_k
  block_expert, slotv = _route(expert_indices)
  ewv = expert_weights.astype(jnp.float32).reshape(_NW, 2 * _T // _NW)
  xg = _sc_dispatch(x, slotv)
  yg = _tc_gemm(xg, w1_weight, w2_weight, block_expert)
  return _sc_combine(yg, slotv, ewv)
